# baseline (speedup 1.0000x reference)
import jax
import jax.numpy as jnp
import numpy as np
from jax.experimental import pallas as pl
from jax.experimental.pallas import tpu as pltpu

_H = _W = 11
_HW = _H * _W
_C = 128
_B = 4096
_CB = 32
_NCHUNK = _B // _CB
_DEPTH = 8


def _spiral_perm() -> np.ndarray:
    cen = _H // 2
    pos = [(cen, cen)]
    for r in range(1, cen + 1):
        pos += [(cen - r, w) for w in range(cen - r + 1, cen + r + 1)]
        pos += [(h, cen + r) for h in range(cen - r + 1, cen + r + 1)]
        pos += [(cen + r, w) for w in range(cen - r, cen + r)]
        pos += [(h, cen - r) for h in range(cen - r, cen + r)]
    return np.array([h * _W + w for h, w in pos], dtype=np.int64)


_P = np.zeros((_HW, _HW), dtype=np.float32)
_P[np.arange(_HW), _spiral_perm()] = 1.0


def _body(p_ref, x_hbm, o_hbm, ibufs, obufs, isems, osems):
    def in_copy(i):
        s = i % _DEPTH
        return pltpu.make_async_copy(
            x_hbm.at[pl.ds(i * _CB, _CB)], ibufs.at[s], isems.at[s])

    def out_copy(i):
        s = i % _DEPTH
        return pltpu.make_async_copy(
            obufs.at[s], o_hbm.at[pl.ds(i * _CB, _CB)], osems.at[s])

    for i in range(_DEPTH):
        in_copy(i).start()

    for i in range(_NCHUNK):
        s = i % _DEPTH
        in_copy(i).wait()
        if i >= _DEPTH:
            out_copy(i - _DEPTH).wait()
        xb = ibufs[s]
        xm = xb.reshape(_CB * _C, _HW)
        ym = jax.lax.dot_general(
            xm, p_ref[...], (((1,), (1,)), ((), ())),
            preferred_element_type=jnp.float32)
        obufs[s] = jnp.transpose(ym.reshape(_CB, _C, _HW), (0, 2, 1))
        out_copy(i).start()
        if i + _DEPTH < _NCHUNK:
            in_copy(i + _DEPTH).start()

    for i in range(_NCHUNK - _DEPTH, _NCHUNK):
        out_copy(i).wait()


@jax.jit
def kernel(x):
    xr = x.reshape(_B, _C, _HW)
    return pl.pallas_call(
        _body,
        in_specs=[
            pl.BlockSpec(memory_space=pltpu.VMEM),
            pl.BlockSpec(memory_space=pltpu.HBM),
        ],
        out_specs=pl.BlockSpec(memory_space=pltpu.HBM),
        out_shape=jax.ShapeDtypeStruct((_B, _HW, _C), x.dtype),
        scratch_shapes=[
            pltpu.VMEM((_DEPTH, _CB, _C, _HW), jnp.float32),
            pltpu.VMEM((_DEPTH, _CB, _HW, _C), jnp.float32),
            pltpu.SemaphoreType.DMA((_DEPTH,)),
            pltpu.SemaphoreType.DMA((_DEPTH,)),
        ],
    )(jnp.asarray(_P), xr)
